# Initial kernel scaffold; baseline (speedup 1.0000x reference)
import jax
import jax.numpy as jnp
from jax.experimental import pallas as pl


def kernel(nmt_table, i2t_wemb, maps_i2t, maps_nmt):
    raise NotImplementedError("write your pallas kernel here")


# speedup vs baseline: 1.1815x; 1.1815x over previous
import functools

import jax
import jax.numpy as jnp
from jax import lax
from jax.experimental import pallas as pl
from jax.experimental.pallas import tpu as pltpu
from jax.experimental.pallas import tpu_sc as plsc

_D = 64
_M = 100000
_NC = 2
_NS = 16
_NW = _NC * _NS
_CHUNK = 3200
_SUB = 80
_NSUB = _CHUNK // _SUB


def _sc_loss_partials(nmt_table, i2t_wemb, maps_i2t, maps_nmt):
    mesh = plsc.VectorSubcoreMesh(core_axis_name="c", subcore_axis_name="s")

    @functools.partial(
        pl.kernel,
        mesh=mesh,
        out_type=jax.ShapeDtypeStruct((_NW, 16), jnp.float32),
        compiler_params=pltpu.CompilerParams(use_tc_tiling_on_sc=False),
        scratch_types=[
            pltpu.VMEM((_CHUNK,), jnp.int32),
            pltpu.VMEM((_CHUNK,), jnp.int32),
            pltpu.VMEM((_SUB, _D), jnp.float32),
            pltpu.VMEM((_SUB, _D), jnp.float32),
            pltpu.VMEM((16,), jnp.float32),
            pltpu.SemaphoreType.DMA,
            pltpu.SemaphoreType.DMA,
        ],
    )
    def k(nmt_hbm, wemb_hbm, mi_hbm, mn_hbm, out_hbm,
          idx_n, idx_i, rows_n, rows_i, acc_v, sem_n, sem_i):
        wid = lax.axis_index("s") * _NC + lax.axis_index("c")
        base = wid * _CHUNK
        pltpu.sync_copy(mn_hbm.at[pl.ds(base, _CHUNK)], idx_n)
        pltpu.sync_copy(mi_hbm.at[pl.ds(base, _CHUNK)], idx_i)
        n_valid = jnp.minimum(_NSUB, (_M - base) // _SUB)

        def sub_body(s, carry):
            off = pl.multiple_of(s * _SUB, 8)
            cp_n = pltpu.async_copy(
                nmt_hbm.at[idx_n.at[pl.ds(off, _SUB)]], rows_n, sem_n)
            cp_i = pltpu.async_copy(
                wemb_hbm.at[idx_i.at[pl.ds(off, _SUB)]], rows_i, sem_i)
            cp_n.wait()
            cp_i.wait()

            def row_body(r, c2):
                b0, b1, b2, b3 = c2
                d0 = rows_n[r, pl.ds(0, 16)] - rows_i[r, pl.ds(0, 16)]
                d1 = rows_n[r, pl.ds(16, 16)] - rows_i[r, pl.ds(16, 16)]
                d2 = rows_n[r, pl.ds(32, 16)] - rows_i[r, pl.ds(32, 16)]
                d3 = rows_n[r, pl.ds(48, 16)] - rows_i[r, pl.ds(48, 16)]
                return (b0 + d0 * d0, b1 + d1 * d1,
                        b2 + d2 * d2, b3 + d3 * d3)

            return lax.fori_loop(0, _SUB, row_body, carry, unroll=2)

        z = jnp.zeros((16,), jnp.float32)
        a0, a1, a2, a3 = lax.fori_loop(0, n_valid, sub_body, (z, z, z, z))
        acc_v[...] = (a0 + a1) + (a2 + a3)
        pltpu.sync_copy(acc_v, out_hbm.at[wid])

    return k(nmt_table, i2t_wemb, maps_i2t, maps_nmt)


def kernel(nmt_table, i2t_wemb, maps_i2t, maps_nmt):
    pad = _NW * _CHUNK - _M
    zpad = jnp.zeros((pad,), jnp.int32)
    mi = jnp.concatenate([maps_i2t, zpad])
    mn = jnp.concatenate([maps_nmt, zpad])
    partials = _sc_loss_partials(nmt_table, i2t_wemb, mi, mn)
    return jnp.sum(partials) / (_M * _D)


# speedup vs baseline: 1.3346x; 1.1295x over previous
import functools

import jax
import jax.numpy as jnp
from jax import lax
from jax.experimental import pallas as pl
from jax.experimental.pallas import tpu as pltpu
from jax.experimental.pallas import tpu_sc as plsc

_D = 64
_M = 100000
_NC = 2
_NS = 16
_NW = _NC * _NS
_CHUNK = 3200
_SUB = 80
_NSUB = _CHUNK // _SUB


def _sc_loss_partials(nmt_table, i2t_wemb, maps_i2t, maps_nmt):
    mesh = plsc.VectorSubcoreMesh(core_axis_name="c", subcore_axis_name="s")

    @functools.partial(
        pl.kernel,
        mesh=mesh,
        out_type=jax.ShapeDtypeStruct((_NW, 16), jnp.float32),
        compiler_params=pltpu.CompilerParams(use_tc_tiling_on_sc=False),
        scratch_types=[
            pltpu.VMEM((_CHUNK,), jnp.int32),
            pltpu.VMEM((_CHUNK,), jnp.int32),
            pltpu.VMEM((_SUB, _D), jnp.float32),
            pltpu.VMEM((_SUB, _D), jnp.float32),
            pltpu.VMEM((_SUB, _D), jnp.float32),
            pltpu.VMEM((_SUB, _D), jnp.float32),
            pltpu.VMEM((16,), jnp.float32),
            pltpu.SemaphoreType.DMA,
            pltpu.SemaphoreType.DMA,
        ],
    )
    def k(nmt_hbm, wemb_hbm, mi_hbm, mn_hbm, out_hbm,
          idx_n, idx_i, rows_n0, rows_n1, rows_i0, rows_i1, acc_v,
          sem0, sem1):
        wid = lax.axis_index("s") * _NC + lax.axis_index("c")
        base = wid * _CHUNK
        pltpu.sync_copy(mn_hbm.at[pl.ds(base, _CHUNK)], idx_n)
        pltpu.sync_copy(mi_hbm.at[pl.ds(base, _CHUNK)], idx_i)
        n_valid = jnp.minimum(_NSUB, (_M - base) // _SUB)

        def issue(s, rn, ri, sem):
            off = pl.multiple_of(s * _SUB, 8)
            pltpu.async_copy(nmt_hbm.at[idx_n.at[pl.ds(off, _SUB)]], rn, sem)
            pltpu.async_copy(wemb_hbm.at[idx_i.at[pl.ds(off, _SUB)]], ri, sem)

        def wait_for(s, rn, ri, sem):
            off = pl.multiple_of(s * _SUB, 8)
            pltpu.make_async_copy(
                nmt_hbm.at[idx_n.at[pl.ds(off, _SUB)]], rn, sem).wait()
            pltpu.make_async_copy(
                wemb_hbm.at[idx_i.at[pl.ds(off, _SUB)]], ri, sem).wait()

        def compute(rn, ri, carry):
            def row_body(r, c2):
                b0, b1, b2, b3 = c2
                d0 = rn[r, pl.ds(0, 16)] - ri[r, pl.ds(0, 16)]
                d1 = rn[r, pl.ds(16, 16)] - ri[r, pl.ds(16, 16)]
                d2 = rn[r, pl.ds(32, 16)] - ri[r, pl.ds(32, 16)]
                d3 = rn[r, pl.ds(48, 16)] - ri[r, pl.ds(48, 16)]
                return (b0 + d0 * d0, b1 + d1 * d1,
                        b2 + d2 * d2, b3 + d3 * d3)

            return lax.fori_loop(0, _SUB, row_body, carry, unroll=4)

        issue(0, rows_n0, rows_i0, sem0)

        def pair_body(h, carry):
            s = 2 * h
            issue(s + 1, rows_n1, rows_i1, sem1)
            wait_for(s, rows_n0, rows_i0, sem0)
            carry = compute(rows_n0, rows_i0, carry)

            @pl.when(s + 2 < n_valid)
            def _():
                issue(s + 2, rows_n0, rows_i0, sem0)

            wait_for(s + 1, rows_n1, rows_i1, sem1)
            return compute(rows_n1, rows_i1, carry)

        z = jnp.zeros((16,), jnp.float32)
        a0, a1, a2, a3 = lax.fori_loop(0, n_valid // 2, pair_body,
                                       (z, z, z, z))
        acc_v[...] = (a0 + a1) + (a2 + a3)
        pltpu.sync_copy(acc_v, out_hbm.at[wid])

    return k(nmt_table, i2t_wemb, maps_i2t, maps_nmt)


def kernel(nmt_table, i2t_wemb, maps_i2t, maps_nmt):
    pad = _NW * _CHUNK - _M
    zpad = jnp.zeros((pad,), jnp.int32)
    mi = jnp.concatenate([maps_i2t, zpad])
    mn = jnp.concatenate([maps_nmt, zpad])
    partials = _sc_loss_partials(nmt_table, i2t_wemb, mi, mn)
    return jnp.sum(partials) / (_M * _D)
